# baseline (speedup 1.0000x reference)
import functools

import jax
import jax.numpy as jnp
from jax import lax
from jax.experimental import pallas as pl
from jax.experimental.pallas import tpu as pltpu
from jax.experimental.pallas import tpu_sc as plsc

_NUM_EXPERT = 64
_D = 768
_CAP = 128
_SLOTS = _NUM_EXPERT * _CAP
_NW = 32
_NC = 2


def _gate_body(x_ref, gw_ref, gb_ref, g0_ref, e0_ref, e1_ref):
    x = x_ref[...]
    logits = jnp.dot(x, gw_ref[...], preferred_element_type=jnp.float32)
    logits = logits + gb_ref[0, 0, :][None, :]
    B = logits.shape[0]
    cols = lax.broadcasted_iota(jnp.int32, logits.shape, 1)
    v0 = jnp.max(logits, axis=1)
    e0 = jnp.min(jnp.where(logits == v0[:, None], cols, _NUM_EXPERT), axis=1)
    masked = jnp.where(cols == e0[:, None], -jnp.inf, logits)
    v1 = jnp.max(masked, axis=1)
    e1 = jnp.min(jnp.where((masked == v1[:, None]) & (cols != e0[:, None]),
                           cols, _NUM_EXPERT), axis=1)
    g0_ref[...] = 1.0 / (1.0 + jnp.exp(v1 - v0))
    e0_ref[...] = e0
    e1_ref[...] = e1


def _gate(x, gate_w, gate_b):
    N = x.shape[0]
    B = 1024
    gb3 = gate_b.reshape(1, 1, _NUM_EXPERT)
    return pl.pallas_call(
        _gate_body,
        grid=(N // B,),
        in_specs=[
            pl.BlockSpec((B, _D), lambda i: (i, 0)),
            pl.BlockSpec((_D, _NUM_EXPERT), lambda i: (0, 0)),
            pl.BlockSpec((1, 1, _NUM_EXPERT), lambda i: (0, 0, 0)),
        ],
        out_specs=[
            pl.BlockSpec((B,), lambda i: (i,)),
            pl.BlockSpec((B,), lambda i: (i,)),
            pl.BlockSpec((B,), lambda i: (i,)),
        ],
        out_shape=[
            jax.ShapeDtypeStruct((N,), jnp.float32),
            jax.ShapeDtypeStruct((N,), jnp.int32),
            jax.ShapeDtypeStruct((N,), jnp.int32),
        ],
    )(x, gate_w, gb3)


def _expert_body(disp_ref, w_ref, wgt_ref, b_ref, out_ref):
    i = pl.program_id(0)
    d = disp_ref[...]
    y = jnp.dot(d, w_ref[0], preferred_element_type=jnp.float32)
    y = y + b_ref[0, 0, :][None, :]
    delta = y - d
    wrow = wgt_ref[0, 0, :]
    r = lax.broadcasted_iota(jnp.int32, (_CAP, _CAP), 0)
    c = lax.broadcasted_iota(jnp.int32, (_CAP, _CAP), 1)
    diag = jnp.where(r == c, jnp.broadcast_to(wrow[None, :], (_CAP, _CAP)), 0.0)
    scaled = jnp.dot(diag, delta, preferred_element_type=jnp.float32)
    out_ref[...] = jnp.where(i == _NUM_EXPERT, 0.0, scaled)


def _expert(disp, expert_w, wgt_tbl, expert_b):
    wgt3 = wgt_tbl.reshape(_NUM_EXPERT, 1, _CAP)
    b3 = expert_b.reshape(_NUM_EXPERT, 1, _D)
    last = _NUM_EXPERT - 1
    return pl.pallas_call(
        _expert_body,
        grid=(_NUM_EXPERT + 1,),
        in_specs=[
            pl.BlockSpec((_CAP, _D), lambda i: (jnp.minimum(i, last), 0)),
            pl.BlockSpec((1, _D, _D), lambda i: (jnp.minimum(i, last), 0, 0)),
            pl.BlockSpec((1, 1, _CAP), lambda i: (jnp.minimum(i, last), 0, 0)),
            pl.BlockSpec((1, 1, _D), lambda i: (jnp.minimum(i, last), 0, 0)),
        ],
        out_specs=pl.BlockSpec((_CAP, _D), lambda i: (i, 0)),
        out_shape=jax.ShapeDtypeStruct(((_NUM_EXPERT + 1) * _CAP, _D),
                                       jnp.float32),
    )(disp, expert_w, wgt3, b3)


def _sc_gather(x, idx_tbl):
    N, D = x.shape
    per_w = _SLOTS // _NW
    CH = 64
    mesh = plsc.VectorSubcoreMesh(core_axis_name="c", subcore_axis_name="s")

    @functools.partial(
        pl.kernel, mesh=mesh,
        out_type=jax.ShapeDtypeStruct((_SLOTS, D), jnp.float32),
        scratch_types=[
            pltpu.VMEM((CH,), jnp.int32),
            pltpu.VMEM((CH, D), jnp.float32),
            pltpu.SemaphoreType.DMA,
        ],
    )
    def k(x_hbm, idx_hbm, out_hbm, idx_v, rows_v, sem):
        wid = lax.axis_index("s") * _NC + lax.axis_index("c")

        def body(c, carry):
            base = wid * per_w + c * CH
            pltpu.sync_copy(idx_hbm.at[pl.ds(base, CH)], idx_v)
            pltpu.async_copy(x_hbm.at[idx_v], rows_v, sem).wait()
            pltpu.sync_copy(rows_v, out_hbm.at[pl.ds(base, CH)])
            return carry

        lax.fori_loop(0, per_w // CH, body, 0)

    return k(x, idx_tbl)


def _sc_gather_pair(delta, p0, p1):
    N = p0.shape[0]
    D = delta.shape[1]
    per_w = N // _NW
    CH = 64
    mesh = plsc.VectorSubcoreMesh(core_axis_name="c", subcore_axis_name="s")

    @functools.partial(
        pl.kernel, mesh=mesh,
        out_type=[
            jax.ShapeDtypeStruct((N, D), jnp.float32),
            jax.ShapeDtypeStruct((N, D), jnp.float32),
        ],
        scratch_types=[
            pltpu.VMEM((CH,), jnp.int32),
            pltpu.VMEM((CH,), jnp.int32),
            pltpu.VMEM((CH, D), jnp.float32),
            pltpu.VMEM((CH, D), jnp.float32),
            pltpu.SemaphoreType.DMA,
        ],
    )
    def k(delta_hbm, p0_hbm, p1_hbm, d0_hbm, d1_hbm,
          p0v, p1v, d0v, d1v, sem):
        wid = lax.axis_index("s") * _NC + lax.axis_index("c")

        def chunk(c, carry):
            tok0 = wid * per_w + c * CH
            pltpu.sync_copy(p0_hbm.at[pl.ds(tok0, CH)], p0v)
            pltpu.sync_copy(p1_hbm.at[pl.ds(tok0, CH)], p1v)
            cp0 = pltpu.async_copy(delta_hbm.at[p0v], d0v, sem)
            cp1 = pltpu.async_copy(delta_hbm.at[p1v], d1v, sem)
            cp0.wait()
            cp1.wait()
            pltpu.sync_copy(d0v, d0_hbm.at[pl.ds(tok0, CH)])
            pltpu.sync_copy(d1v, d1_hbm.at[pl.ds(tok0, CH)])
            return carry

        lax.fori_loop(0, per_w // CH, chunk, 0)

    return k(delta, p0, p1)


def _combine_body(x_ref, d0_ref, d1_ref, o_ref):
    o_ref[...] = x_ref[...] + d0_ref[...] + d1_ref[...]


def _combine(x, d0, d1):
    N, D = x.shape
    B = 1024
    spec = pl.BlockSpec((B, D), lambda i: (i, 0))
    return pl.pallas_call(
        _combine_body,
        grid=(N // B,),
        in_specs=[spec, spec, spec],
        out_specs=spec,
        out_shape=jax.ShapeDtypeStruct((N, D), jnp.float32),
    )(x, d0, d1)


@jax.jit
def kernel(moe_inp, gate_w, gate_b, expert_w, expert_b):
    x = moe_inp
    N = x.shape[0]
    n_slots = N * 2

    g0, e0, e1 = _gate(x, gate_w, gate_b)

    slot_expert = jnp.stack([e0, e1], axis=1).reshape(-1)
    slot_score = jnp.stack([g0, g0], axis=1).reshape(-1)
    sort_key = slot_expert.astype(jnp.float32) * 4.0 - slot_score
    order = jnp.argsort(sort_key)
    sorted_expert = slot_expert[order]
    counts = jnp.bincount(slot_expert, length=_NUM_EXPERT)
    starts = jnp.concatenate(
        [jnp.zeros((1,), counts.dtype), jnp.cumsum(counts)[:-1]])
    rank = jnp.arange(n_slots, dtype=jnp.int32) - starts[sorted_expert]
    keep_sorted = rank < _CAP
    loc_sorted = jnp.where(keep_sorted,
                           sorted_expert * _CAP + rank,
                           _SLOTS).astype(jnp.int32)

    pos = jnp.zeros((n_slots,), jnp.int32).at[order].set(loc_sorted)
    p0 = pos[0::2]
    p1 = pos[1::2]

    tok_sorted = (order // 2).astype(jnp.int32)
    g_sorted = jnp.where(order % 2 == 0, g0[tok_sorted], 1.0 - g0[tok_sorted])
    idx_tbl = jnp.zeros((_SLOTS,), jnp.int32).at[loc_sorted].set(
        tok_sorted, mode='drop')
    wgt_tbl = jnp.zeros((_SLOTS,), jnp.float32).at[loc_sorted].set(
        g_sorted, mode='drop')

    disp = _sc_gather(x, idx_tbl)
    delta = _expert(disp, expert_w, wgt_tbl, expert_b)
    d0, d1 = _sc_gather_pair(delta, p0, p1)
    return _combine(x, d0, d1)


# speedup vs baseline: 2.1406x; 2.1406x over previous
import functools

import jax
import jax.numpy as jnp
from jax import lax
from jax.experimental import pallas as pl
from jax.experimental.pallas import tpu as pltpu
from jax.experimental.pallas import tpu_sc as plsc

_NUM_EXPERT = 64
_D = 768
_CAP = 128
_SLOTS = _NUM_EXPERT * _CAP
_NW = 32
_NC = 2


def _gate_body(x_ref, gw_ref, gb_ref, g0_ref, e0_ref, e1_ref):
    x = x_ref[...]
    logits = jnp.dot(x, gw_ref[...], preferred_element_type=jnp.float32)
    logits = logits + gb_ref[0, 0, :][None, :]
    B = logits.shape[0]
    cols = lax.broadcasted_iota(jnp.int32, logits.shape, 1)
    v0 = jnp.max(logits, axis=1)
    e0 = jnp.min(jnp.where(logits == v0[:, None], cols, _NUM_EXPERT), axis=1)
    masked = jnp.where(cols == e0[:, None], -jnp.inf, logits)
    v1 = jnp.max(masked, axis=1)
    e1 = jnp.min(jnp.where((masked == v1[:, None]) & (cols != e0[:, None]),
                           cols, _NUM_EXPERT), axis=1)
    g0_ref[...] = 1.0 / (1.0 + jnp.exp(v1 - v0))
    e0_ref[...] = e0
    e1_ref[...] = e1


def _gate(x, gate_w, gate_b):
    N = x.shape[0]
    B = 1024
    gb3 = gate_b.reshape(1, 1, _NUM_EXPERT)
    return pl.pallas_call(
        _gate_body,
        grid=(N // B,),
        in_specs=[
            pl.BlockSpec((B, _D), lambda i: (i, 0)),
            pl.BlockSpec((_D, _NUM_EXPERT), lambda i: (0, 0)),
            pl.BlockSpec((1, 1, _NUM_EXPERT), lambda i: (0, 0, 0)),
        ],
        out_specs=[
            pl.BlockSpec((B,), lambda i: (i,)),
            pl.BlockSpec((B,), lambda i: (i,)),
            pl.BlockSpec((B,), lambda i: (i,)),
        ],
        out_shape=[
            jax.ShapeDtypeStruct((N,), jnp.float32),
            jax.ShapeDtypeStruct((N,), jnp.int32),
            jax.ShapeDtypeStruct((N,), jnp.int32),
        ],
    )(x, gate_w, gb3)


def _expert_body(disp_ref, w_ref, wgt_ref, b_ref, out_ref):
    i = pl.program_id(0)
    d = disp_ref[...]
    y = jnp.dot(d, w_ref[0], preferred_element_type=jnp.float32)
    y = y + b_ref[0, 0, :][None, :]
    delta = y - d
    wrow = wgt_ref[0, 0, :]
    r = lax.broadcasted_iota(jnp.int32, (_CAP, _CAP), 0)
    c = lax.broadcasted_iota(jnp.int32, (_CAP, _CAP), 1)
    diag = jnp.where(r == c, jnp.broadcast_to(wrow[None, :], (_CAP, _CAP)), 0.0)
    scaled = jnp.dot(diag, delta, preferred_element_type=jnp.float32)
    out_ref[...] = jnp.where(i == _NUM_EXPERT, 0.0, scaled)


def _expert(disp, expert_w, wgt_tbl, expert_b):
    wgt3 = wgt_tbl.reshape(_NUM_EXPERT, 1, _CAP)
    b3 = expert_b.reshape(_NUM_EXPERT, 1, _D)
    last = _NUM_EXPERT - 1
    return pl.pallas_call(
        _expert_body,
        grid=(_NUM_EXPERT + 1,),
        in_specs=[
            pl.BlockSpec((_CAP, _D), lambda i: (jnp.minimum(i, last), 0)),
            pl.BlockSpec((1, _D, _D), lambda i: (jnp.minimum(i, last), 0, 0)),
            pl.BlockSpec((1, 1, _CAP), lambda i: (jnp.minimum(i, last), 0, 0)),
            pl.BlockSpec((1, 1, _D), lambda i: (jnp.minimum(i, last), 0, 0)),
        ],
        out_specs=pl.BlockSpec((_CAP, _D), lambda i: (i, 0)),
        out_shape=jax.ShapeDtypeStruct(((_NUM_EXPERT + 1) * _CAP, _D),
                                       jnp.float32),
    )(disp, expert_w, wgt3, b3)


def _sc_gather(x, idx_tbl):
    N, D = x.shape
    per_w = _SLOTS // _NW
    CH = 64
    mesh = plsc.VectorSubcoreMesh(core_axis_name="c", subcore_axis_name="s")

    @functools.partial(
        pl.kernel, mesh=mesh,
        out_type=jax.ShapeDtypeStruct((_SLOTS, D), jnp.float32),
        scratch_types=[
            pltpu.VMEM((CH,), jnp.int32),
            pltpu.VMEM((CH, D), jnp.float32),
            pltpu.SemaphoreType.DMA,
        ],
    )
    def k(x_hbm, idx_hbm, out_hbm, idx_v, rows_v, sem):
        wid = lax.axis_index("s") * _NC + lax.axis_index("c")

        def body(c, carry):
            base = wid * per_w + c * CH
            pltpu.sync_copy(idx_hbm.at[pl.ds(base, CH)], idx_v)
            pltpu.async_copy(x_hbm.at[idx_v], rows_v, sem).wait()
            pltpu.sync_copy(rows_v, out_hbm.at[pl.ds(base, CH)])
            return carry

        lax.fori_loop(0, per_w // CH, body, 0)

    return k(x, idx_tbl)


def _sc_gather_pair(delta, p0, p1):
    N = p0.shape[0]
    D = delta.shape[1]
    per_w = N // _NW
    CH = 64
    mesh = plsc.VectorSubcoreMesh(core_axis_name="c", subcore_axis_name="s")

    @functools.partial(
        pl.kernel, mesh=mesh,
        out_type=[
            jax.ShapeDtypeStruct((N, D), jnp.float32),
            jax.ShapeDtypeStruct((N, D), jnp.float32),
        ],
        scratch_types=[
            pltpu.VMEM((CH,), jnp.int32),
            pltpu.VMEM((CH,), jnp.int32),
            pltpu.VMEM((CH, D), jnp.float32),
            pltpu.VMEM((CH, D), jnp.float32),
            pltpu.SemaphoreType.DMA,
        ],
    )
    def k(delta_hbm, p0_hbm, p1_hbm, d0_hbm, d1_hbm,
          p0v, p1v, d0v, d1v, sem):
        wid = lax.axis_index("s") * _NC + lax.axis_index("c")

        def chunk(c, carry):
            tok0 = wid * per_w + c * CH
            pltpu.sync_copy(p0_hbm.at[pl.ds(tok0, CH)], p0v)
            pltpu.sync_copy(p1_hbm.at[pl.ds(tok0, CH)], p1v)
            cp0 = pltpu.async_copy(delta_hbm.at[p0v], d0v, sem)
            cp1 = pltpu.async_copy(delta_hbm.at[p1v], d1v, sem)
            cp0.wait()
            cp1.wait()
            pltpu.sync_copy(d0v, d0_hbm.at[pl.ds(tok0, CH)])
            pltpu.sync_copy(d1v, d1_hbm.at[pl.ds(tok0, CH)])
            return carry

        lax.fori_loop(0, per_w // CH, chunk, 0)

    return k(delta, p0, p1)


def _combine_body(x_ref, d0_ref, d1_ref, o_ref):
    o_ref[...] = x_ref[...] + d0_ref[...] + d1_ref[...]


def _combine(x, d0, d1):
    N, D = x.shape
    B = 1024
    spec = pl.BlockSpec((B, D), lambda i: (i, 0))
    return pl.pallas_call(
        _combine_body,
        grid=(N // B,),
        in_specs=[spec, spec, spec],
        out_specs=spec,
        out_shape=jax.ShapeDtypeStruct((N, D), jnp.float32),
    )(x, d0, d1)


@jax.jit
def kernel(moe_inp, gate_w, gate_b, expert_w, expert_b):
    x = moe_inp
    N = x.shape[0]
    n_slots = N * 2

    g0, e0, e1 = _gate(x, gate_w, gate_b)

    slot_expert = jnp.stack([e0, e1], axis=1).reshape(-1)
    slot_score = jnp.stack([g0, g0], axis=1).reshape(-1)
    sort_key = slot_expert.astype(jnp.float32) * 4.0 - slot_score
    order = jnp.argsort(sort_key)
    sorted_expert = slot_expert[order]
    counts = jnp.bincount(slot_expert, length=_NUM_EXPERT)
    starts = jnp.concatenate(
        [jnp.zeros((1,), counts.dtype), jnp.cumsum(counts)[:-1]])
    rank = jnp.arange(n_slots, dtype=jnp.int32) - starts[sorted_expert]
    keep_sorted = rank < _CAP
    tok_sorted = (order // 2).astype(jnp.int32)
    loc_sorted = jnp.where(keep_sorted,
                           sorted_expert * _CAP + rank,
                           _SLOTS + (tok_sorted % _CAP)).astype(jnp.int32)
    loc_tbl = jnp.where(keep_sorted,
                        sorted_expert * _CAP + rank,
                        n_slots).astype(jnp.int32)

    pos = jnp.zeros((n_slots,), jnp.int32).at[order].set(loc_sorted)
    p0 = pos[0::2]
    p1 = pos[1::2]

    g_sorted = jnp.where(order % 2 == 0, g0[tok_sorted], 1.0 - g0[tok_sorted])
    idx_tbl = jnp.zeros((_SLOTS,), jnp.int32).at[loc_tbl].set(
        tok_sorted, mode='drop')
    wgt_tbl = jnp.zeros((_SLOTS,), jnp.float32).at[loc_tbl].set(
        g_sorted, mode='drop')

    disp = _sc_gather(x, idx_tbl)
    delta = _expert(disp, expert_w, wgt_tbl, expert_b)
    d0, d1 = _sc_gather_pair(delta, p0, p1)
    return _combine(x, d0, d1)


# speedup vs baseline: 3.0336x; 1.4172x over previous
import functools

import jax
import jax.numpy as jnp
from jax import lax
from jax.experimental import pallas as pl
from jax.experimental.pallas import tpu as pltpu
from jax.experimental.pallas import tpu_sc as plsc

_NUM_EXPERT = 64
_D = 768
_CAP = 128
_SLOTS = _NUM_EXPERT * _CAP
_NW = 32
_NC = 2


def _gate_body(x_ref, gw_ref, gb_ref, g0_ref, e0_ref, e1_ref):
    x = x_ref[...]
    logits = jnp.dot(x, gw_ref[...], preferred_element_type=jnp.float32)
    logits = logits + gb_ref[0, 0, :][None, :]
    B = logits.shape[0]
    cols = lax.broadcasted_iota(jnp.int32, logits.shape, 1)
    v0 = jnp.max(logits, axis=1)
    e0 = jnp.min(jnp.where(logits == v0[:, None], cols, _NUM_EXPERT), axis=1)
    masked = jnp.where(cols == e0[:, None], -jnp.inf, logits)
    v1 = jnp.max(masked, axis=1)
    e1 = jnp.min(jnp.where((masked == v1[:, None]) & (cols != e0[:, None]),
                           cols, _NUM_EXPERT), axis=1)
    g0_ref[...] = 1.0 / (1.0 + jnp.exp(v1 - v0))
    e0_ref[...] = e0
    e1_ref[...] = e1


def _gate(x, gate_w, gate_b):
    N = x.shape[0]
    B = 1024
    gb3 = gate_b.reshape(1, 1, _NUM_EXPERT)
    return pl.pallas_call(
        _gate_body,
        grid=(N // B,),
        in_specs=[
            pl.BlockSpec((B, _D), lambda i: (i, 0)),
            pl.BlockSpec((_D, _NUM_EXPERT), lambda i: (0, 0)),
            pl.BlockSpec((1, 1, _NUM_EXPERT), lambda i: (0, 0, 0)),
        ],
        out_specs=[
            pl.BlockSpec((B,), lambda i: (i,)),
            pl.BlockSpec((B,), lambda i: (i,)),
            pl.BlockSpec((B,), lambda i: (i,)),
        ],
        out_shape=[
            jax.ShapeDtypeStruct((N,), jnp.float32),
            jax.ShapeDtypeStruct((N,), jnp.int32),
            jax.ShapeDtypeStruct((N,), jnp.int32),
        ],
    )(x, gate_w, gb3)


def _expert_body(disp_ref, w_ref, wgt_ref, b_ref, out_ref):
    i = pl.program_id(0)
    d = disp_ref[...]
    y = jnp.dot(d, w_ref[0], preferred_element_type=jnp.float32)
    y = y + b_ref[0, 0, :][None, :]
    delta = y - d
    wrow = wgt_ref[0, 0, :]
    r = lax.broadcasted_iota(jnp.int32, (_CAP, _CAP), 0)
    c = lax.broadcasted_iota(jnp.int32, (_CAP, _CAP), 1)
    diag = jnp.where(r == c, jnp.broadcast_to(wrow[None, :], (_CAP, _CAP)), 0.0)
    scaled = jnp.dot(diag, delta, preferred_element_type=jnp.float32)
    out_ref[...] = jnp.where(i == _NUM_EXPERT, 0.0, scaled)


def _expert(disp, expert_w, wgt_tbl, expert_b):
    wgt3 = wgt_tbl.reshape(_NUM_EXPERT, 1, _CAP)
    b3 = expert_b.reshape(_NUM_EXPERT, 1, _D)
    last = _NUM_EXPERT - 1
    return pl.pallas_call(
        _expert_body,
        grid=(_NUM_EXPERT + 1,),
        in_specs=[
            pl.BlockSpec((_CAP, _D), lambda i: (jnp.minimum(i, last), 0)),
            pl.BlockSpec((1, _D, _D), lambda i: (jnp.minimum(i, last), 0, 0)),
            pl.BlockSpec((1, 1, _CAP), lambda i: (jnp.minimum(i, last), 0, 0)),
            pl.BlockSpec((1, 1, _D), lambda i: (jnp.minimum(i, last), 0, 0)),
        ],
        out_specs=pl.BlockSpec((_CAP, _D), lambda i: (i, 0)),
        out_shape=jax.ShapeDtypeStruct(((_NUM_EXPERT + 1) * _CAP, _D),
                                       jnp.float32),
    )(disp, expert_w, wgt3, b3)


def _sc_gather(x, idx_tbl):
    N, D = x.shape
    per_w = _SLOTS // _NW
    CH = 64
    mesh = plsc.VectorSubcoreMesh(core_axis_name="c", subcore_axis_name="s")

    @functools.partial(
        pl.kernel, mesh=mesh,
        out_type=jax.ShapeDtypeStruct((_SLOTS, D), jnp.float32),
        scratch_types=[
            pltpu.VMEM((CH,), jnp.int32),
            pltpu.VMEM((CH, D), jnp.float32),
            pltpu.SemaphoreType.DMA,
        ],
    )
    def k(x_hbm, idx_hbm, out_hbm, idx_v, rows_v, sem):
        wid = lax.axis_index("s") * _NC + lax.axis_index("c")

        def body(c, carry):
            base = wid * per_w + c * CH
            pltpu.sync_copy(idx_hbm.at[pl.ds(base, CH)], idx_v)
            pltpu.async_copy(x_hbm.at[idx_v], rows_v, sem).wait()
            pltpu.sync_copy(rows_v, out_hbm.at[pl.ds(base, CH)])
            return carry

        lax.fori_loop(0, per_w // CH, body, 0)

    return k(x, idx_tbl)


def _sc_gather_pair(delta, p0, p1):
    N = p0.shape[0]
    D = delta.shape[1]
    per_w = N // _NW
    CH = 64
    mesh = plsc.VectorSubcoreMesh(core_axis_name="c", subcore_axis_name="s")

    @functools.partial(
        pl.kernel, mesh=mesh,
        out_type=[
            jax.ShapeDtypeStruct((N, D), jnp.float32),
            jax.ShapeDtypeStruct((N, D), jnp.float32),
        ],
        scratch_types=[
            pltpu.VMEM((CH,), jnp.int32),
            pltpu.VMEM((CH,), jnp.int32),
            pltpu.VMEM((CH, D), jnp.float32),
            pltpu.VMEM((CH, D), jnp.float32),
            pltpu.SemaphoreType.DMA,
        ],
    )
    def k(delta_hbm, p0_hbm, p1_hbm, d0_hbm, d1_hbm,
          p0v, p1v, d0v, d1v, sem):
        wid = lax.axis_index("s") * _NC + lax.axis_index("c")

        def chunk(c, carry):
            tok0 = wid * per_w + c * CH
            pltpu.sync_copy(p0_hbm.at[pl.ds(tok0, CH)], p0v)
            pltpu.sync_copy(p1_hbm.at[pl.ds(tok0, CH)], p1v)
            cp0 = pltpu.async_copy(delta_hbm.at[p0v], d0v, sem)
            cp1 = pltpu.async_copy(delta_hbm.at[p1v], d1v, sem)
            cp0.wait()
            cp1.wait()
            pltpu.sync_copy(d0v, d0_hbm.at[pl.ds(tok0, CH)])
            pltpu.sync_copy(d1v, d1_hbm.at[pl.ds(tok0, CH)])
            return carry

        lax.fori_loop(0, per_w // CH, chunk, 0)

    return k(delta, p0, p1)


def _combine_body(x_ref, d0_ref, d1_ref, o_ref):
    o_ref[...] = x_ref[...] + d0_ref[...] + d1_ref[...]


def _combine(x, d0, d1):
    N, D = x.shape
    B = 1024
    spec = pl.BlockSpec((B, D), lambda i: (i, 0))
    return pl.pallas_call(
        _combine_body,
        grid=(N // B,),
        in_specs=[spec, spec, spec],
        out_specs=spec,
        out_shape=jax.ShapeDtypeStruct((N, D), jnp.float32),
    )(x, d0, d1)


def _route(g0, e0, e1):
    N = g0.shape[0]
    n_slots = N * 2
    slot_expert = jnp.stack([e0, e1], axis=1).reshape(-1)
    slot_score = jnp.stack([g0, g0], axis=1).reshape(-1)
    sort_key = slot_expert.astype(jnp.float32) * 4.0 - slot_score
    order = jnp.argsort(sort_key).astype(jnp.int32)
    inv = jnp.argsort(order).astype(jnp.int32)
    sorted_expert = slot_expert[order]

    e_ids = jnp.arange(_NUM_EXPERT + 1, dtype=jnp.int32)
    starts_ext = jnp.sum(
        (slot_expert[None, :] < e_ids[:, None]).astype(jnp.int32), axis=1)
    starts = starts_ext[:_NUM_EXPERT]
    counts = starts_ext[1:] - starts

    rank = jnp.arange(n_slots, dtype=jnp.int32) - starts[sorted_expert]
    keep_sorted = rank < _CAP
    tok_sorted = (order // 2).astype(jnp.int32)
    loc_sorted = jnp.where(keep_sorted,
                           sorted_expert * _CAP + rank,
                           _SLOTS + (tok_sorted % _CAP)).astype(jnp.int32)

    pos = loc_sorted[inv]
    p0 = pos[0::2]
    p1 = pos[1::2]

    g_sorted = jnp.where(order % 2 == 0, g0[tok_sorted], 1.0 - g0[tok_sorted])
    pe = jnp.arange(_SLOTS, dtype=jnp.int32) // _CAP
    pr = jnp.arange(_SLOTS, dtype=jnp.int32) % _CAP
    j_p = jnp.minimum(starts[pe] + pr, n_slots - 1)
    valid = pr < counts[pe]
    idx_tbl = jnp.where(valid, tok_sorted[j_p], 0)
    wgt_tbl = jnp.where(valid, g_sorted[j_p], 0.0)
    return p0, p1, idx_tbl, wgt_tbl


@jax.jit
def kernel(moe_inp, gate_w, gate_b, expert_w, expert_b):
    x = moe_inp
    g0, e0, e1 = _gate(x, gate_w, gate_b)
    p0, p1, idx_tbl, wgt_tbl = _route(g0, e0, e1)
    disp = _sc_gather(x, idx_tbl)
    delta = _expert(disp, expert_w, wgt_tbl, expert_b)
    d0, d1 = _sc_gather_pair(delta, p0, p1)
    return _combine(x, d0, d1)


# speedup vs baseline: 5.0397x; 1.6613x over previous
import functools

import jax
import jax.numpy as jnp
from jax import lax
from jax.experimental import pallas as pl
from jax.experimental.pallas import tpu as pltpu
from jax.experimental.pallas import tpu_sc as plsc

_NUM_EXPERT = 64
_D = 768
_CAP = 128
_SLOTS = _NUM_EXPERT * _CAP
_NW = 32
_NC = 2


def _gate_body(x_ref, gw_ref, gb_ref, g0_ref, e0_ref, e1_ref):
    x = x_ref[...]
    logits = jnp.dot(x, gw_ref[...], preferred_element_type=jnp.float32)
    logits = logits + gb_ref[0, 0, :][None, :]
    B = logits.shape[0]
    cols = lax.broadcasted_iota(jnp.int32, logits.shape, 1)
    v0 = jnp.max(logits, axis=1)
    e0 = jnp.min(jnp.where(logits == v0[:, None], cols, _NUM_EXPERT), axis=1)
    masked = jnp.where(cols == e0[:, None], -jnp.inf, logits)
    v1 = jnp.max(masked, axis=1)
    e1 = jnp.min(jnp.where((masked == v1[:, None]) & (cols != e0[:, None]),
                           cols, _NUM_EXPERT), axis=1)
    g0_ref[...] = 1.0 / (1.0 + jnp.exp(v1 - v0))
    e0_ref[...] = e0
    e1_ref[...] = e1


def _gate(x, gate_w, gate_b):
    N = x.shape[0]
    B = 1024
    gb3 = gate_b.reshape(1, 1, _NUM_EXPERT)
    return pl.pallas_call(
        _gate_body,
        grid=(N // B,),
        in_specs=[
            pl.BlockSpec((B, _D), lambda i: (i, 0)),
            pl.BlockSpec((_D, _NUM_EXPERT), lambda i: (0, 0)),
            pl.BlockSpec((1, 1, _NUM_EXPERT), lambda i: (0, 0, 0)),
        ],
        out_specs=[
            pl.BlockSpec((B,), lambda i: (i,)),
            pl.BlockSpec((B,), lambda i: (i,)),
            pl.BlockSpec((B,), lambda i: (i,)),
        ],
        out_shape=[
            jax.ShapeDtypeStruct((N,), jnp.float32),
            jax.ShapeDtypeStruct((N,), jnp.int32),
            jax.ShapeDtypeStruct((N,), jnp.int32),
        ],
    )(x, gate_w, gb3)


def _expert_body(disp_ref, w_ref, wgt_ref, b_ref, out_ref):
    i = pl.program_id(0)
    d = disp_ref[...]
    y = jnp.dot(d, w_ref[0], preferred_element_type=jnp.float32)
    y = y + b_ref[0, 0, :][None, :]
    delta = y - d
    wrow = wgt_ref[0, 0, :]
    r = lax.broadcasted_iota(jnp.int32, (_CAP, _CAP), 0)
    c = lax.broadcasted_iota(jnp.int32, (_CAP, _CAP), 1)
    diag = jnp.where(r == c, jnp.broadcast_to(wrow[None, :], (_CAP, _CAP)), 0.0)
    scaled = jnp.dot(diag, delta, preferred_element_type=jnp.float32)
    out_ref[...] = jnp.where(i == _NUM_EXPERT, 0.0, scaled)


def _expert(disp, expert_w, wgt_tbl, expert_b):
    wgt3 = wgt_tbl.reshape(_NUM_EXPERT, 1, _CAP)
    b3 = expert_b.reshape(_NUM_EXPERT, 1, _D)
    last = _NUM_EXPERT - 1
    return pl.pallas_call(
        _expert_body,
        grid=(_NUM_EXPERT + 1,),
        in_specs=[
            pl.BlockSpec((_CAP, _D), lambda i: (jnp.minimum(i, last), 0)),
            pl.BlockSpec((1, _D, _D), lambda i: (jnp.minimum(i, last), 0, 0)),
            pl.BlockSpec((1, 1, _CAP), lambda i: (jnp.minimum(i, last), 0, 0)),
            pl.BlockSpec((1, 1, _D), lambda i: (jnp.minimum(i, last), 0, 0)),
        ],
        out_specs=pl.BlockSpec((_CAP, _D), lambda i: (i, 0)),
        out_shape=jax.ShapeDtypeStruct(((_NUM_EXPERT + 1) * _CAP, _D),
                                       jnp.float32),
    )(disp, expert_w, wgt3, b3)


def _sc_gather(x, idx_tbl):
    N, D = x.shape
    per_w = _SLOTS // _NW
    CH = 64
    mesh = plsc.VectorSubcoreMesh(core_axis_name="c", subcore_axis_name="s")

    @functools.partial(
        pl.kernel, mesh=mesh,
        out_type=jax.ShapeDtypeStruct((_SLOTS, D), jnp.float32),
        scratch_types=[
            pltpu.VMEM((CH,), jnp.int32),
            pltpu.VMEM((CH, D), jnp.float32),
            pltpu.SemaphoreType.DMA,
        ],
    )
    def k(x_hbm, idx_hbm, out_hbm, idx_v, rows_v, sem):
        wid = lax.axis_index("s") * _NC + lax.axis_index("c")

        def body(c, carry):
            base = wid * per_w + c * CH
            pltpu.sync_copy(idx_hbm.at[pl.ds(base, CH)], idx_v)
            pltpu.async_copy(x_hbm.at[idx_v], rows_v, sem).wait()
            pltpu.sync_copy(rows_v, out_hbm.at[pl.ds(base, CH)])
            return carry

        lax.fori_loop(0, per_w // CH, body, 0)

    return k(x, idx_tbl)


def _sc_gather_pair(delta, p0, p1):
    N = p0.shape[0]
    D = delta.shape[1]
    per_w = N // _NW
    CH = 64
    mesh = plsc.VectorSubcoreMesh(core_axis_name="c", subcore_axis_name="s")

    @functools.partial(
        pl.kernel, mesh=mesh,
        out_type=[
            jax.ShapeDtypeStruct((N, D), jnp.float32),
            jax.ShapeDtypeStruct((N, D), jnp.float32),
        ],
        scratch_types=[
            pltpu.VMEM((CH,), jnp.int32),
            pltpu.VMEM((CH,), jnp.int32),
            pltpu.VMEM((CH, D), jnp.float32),
            pltpu.VMEM((CH, D), jnp.float32),
            pltpu.SemaphoreType.DMA,
        ],
    )
    def k(delta_hbm, p0_hbm, p1_hbm, d0_hbm, d1_hbm,
          p0v, p1v, d0v, d1v, sem):
        wid = lax.axis_index("s") * _NC + lax.axis_index("c")

        def chunk(c, carry):
            tok0 = wid * per_w + c * CH
            pltpu.sync_copy(p0_hbm.at[pl.ds(tok0, CH)], p0v)
            pltpu.sync_copy(p1_hbm.at[pl.ds(tok0, CH)], p1v)
            cp0 = pltpu.async_copy(delta_hbm.at[p0v], d0v, sem)
            cp1 = pltpu.async_copy(delta_hbm.at[p1v], d1v, sem)
            cp0.wait()
            cp1.wait()
            pltpu.sync_copy(d0v, d0_hbm.at[pl.ds(tok0, CH)])
            pltpu.sync_copy(d1v, d1_hbm.at[pl.ds(tok0, CH)])
            return carry

        lax.fori_loop(0, per_w // CH, chunk, 0)

    return k(delta, p0, p1)


def _combine_body(x_ref, d0_ref, d1_ref, o_ref):
    o_ref[...] = x_ref[...] + d0_ref[...] + d1_ref[...]


def _combine(x, d0, d1):
    N, D = x.shape
    B = 1024
    spec = pl.BlockSpec((B, D), lambda i: (i, 0))
    return pl.pallas_call(
        _combine_body,
        grid=(N // B,),
        in_specs=[spec, spec, spec],
        out_specs=spec,
        out_shape=jax.ShapeDtypeStruct((N, D), jnp.float32),
    )(x, d0, d1)


def _route(g0, e0, e1):
    N = g0.shape[0]
    n_slots = N * 2
    slot_expert = jnp.stack([e0, e1], axis=1).reshape(-1)
    slot_score = jnp.stack([g0, g0], axis=1).reshape(-1)
    sort_key = slot_expert.astype(jnp.float32) * 4.0 - slot_score
    order = jnp.argsort(sort_key).astype(jnp.int32)
    inv = jnp.argsort(order).astype(jnp.int32)
    sorted_expert = slot_expert[order]

    e_ids = jnp.arange(_NUM_EXPERT + 1, dtype=jnp.int32)
    starts_ext = jnp.sum(
        (slot_expert[None, :] < e_ids[:, None]).astype(jnp.int32), axis=1)
    starts = starts_ext[:_NUM_EXPERT]
    counts = starts_ext[1:] - starts

    rank = jnp.arange(n_slots, dtype=jnp.int32) - starts[sorted_expert]
    keep_sorted = rank < _CAP
    tok_sorted = (order // 2).astype(jnp.int32)
    loc_sorted = jnp.where(keep_sorted,
                           sorted_expert * _CAP + rank,
                           _SLOTS + (tok_sorted % _CAP)).astype(jnp.int32)

    pos = loc_sorted[inv]
    p0 = pos[0::2]
    p1 = pos[1::2]

    slot_g = jnp.stack([g0, 1.0 - g0], axis=1).reshape(-1)
    pr = jnp.arange(_SLOTS, dtype=jnp.int32) % _CAP
    starts_rep = jnp.broadcast_to(
        starts[:, None], (_NUM_EXPERT, _CAP)).reshape(_SLOTS)
    counts_rep = jnp.broadcast_to(
        counts[:, None], (_NUM_EXPERT, _CAP)).reshape(_SLOTS)
    j_p = lax.optimization_barrier(
        jnp.minimum(starts_rep + pr, n_slots - 1))
    valid = pr < counts_rep
    order_jp = order[j_p]
    idx_tbl = jnp.where(valid, order_jp // 2, 0)
    wgt_tbl = jnp.where(valid, slot_g[order_jp], 0.0)
    return p0, p1, idx_tbl, wgt_tbl


@jax.jit
def kernel(moe_inp, gate_w, gate_b, expert_w, expert_b):
    x = moe_inp
    g0, e0, e1 = _gate(x, gate_w, gate_b)
    p0, p1, idx_tbl, wgt_tbl = _route(g0, e0, e1)
    disp = _sc_gather(x, idx_tbl)
    delta = _expert(disp, expert_w, wgt_tbl, expert_b)
    d0, d1 = _sc_gather_pair(delta, p0, p1)
    return _combine(x, d0, d1)


# speedup vs baseline: 5.7162x; 1.1342x over previous
import functools

import jax
import jax.numpy as jnp
from jax import lax
from jax.experimental import pallas as pl
from jax.experimental.pallas import tpu as pltpu
from jax.experimental.pallas import tpu_sc as plsc

_NUM_EXPERT = 64
_D = 768
_CAP = 128
_SLOTS = _NUM_EXPERT * _CAP
_NW = 32
_NC = 2


def _gate_body(x_ref, gw_ref, gb_ref, g0_ref, e0_ref, e1_ref):
    x = x_ref[...]
    logits = jnp.dot(x, gw_ref[...], preferred_element_type=jnp.float32)
    logits = logits + gb_ref[0, 0, :][None, :]
    B = logits.shape[0]
    cols = lax.broadcasted_iota(jnp.int32, logits.shape, 1)
    v0 = jnp.max(logits, axis=1)
    e0 = jnp.min(jnp.where(logits == v0[:, None], cols, _NUM_EXPERT), axis=1)
    masked = jnp.where(cols == e0[:, None], -jnp.inf, logits)
    v1 = jnp.max(masked, axis=1)
    e1 = jnp.min(jnp.where((masked == v1[:, None]) & (cols != e0[:, None]),
                           cols, _NUM_EXPERT), axis=1)
    g0_ref[...] = 1.0 / (1.0 + jnp.exp(v1 - v0))
    e0_ref[...] = e0
    e1_ref[...] = e1


def _gate(x, gate_w, gate_b):
    N = x.shape[0]
    B = 1024
    gb3 = gate_b.reshape(1, 1, _NUM_EXPERT)
    return pl.pallas_call(
        _gate_body,
        grid=(N // B,),
        in_specs=[
            pl.BlockSpec((B, _D), lambda i: (i, 0)),
            pl.BlockSpec((_D, _NUM_EXPERT), lambda i: (0, 0)),
            pl.BlockSpec((1, 1, _NUM_EXPERT), lambda i: (0, 0, 0)),
        ],
        out_specs=[
            pl.BlockSpec((B,), lambda i: (i,)),
            pl.BlockSpec((B,), lambda i: (i,)),
            pl.BlockSpec((B,), lambda i: (i,)),
        ],
        out_shape=[
            jax.ShapeDtypeStruct((N,), jnp.float32),
            jax.ShapeDtypeStruct((N,), jnp.int32),
            jax.ShapeDtypeStruct((N,), jnp.int32),
        ],
    )(x, gate_w, gb3)


def _expert_body(disp_ref, w_ref, wgt_ref, b_ref, out_ref):
    i = pl.program_id(0)
    d = disp_ref[...]
    y = jnp.dot(d, w_ref[0], preferred_element_type=jnp.float32)
    y = y + b_ref[0, 0, :][None, :]
    delta = y - d
    wrow = wgt_ref[0, 0, :]
    r = lax.broadcasted_iota(jnp.int32, (_CAP, _CAP), 0)
    c = lax.broadcasted_iota(jnp.int32, (_CAP, _CAP), 1)
    diag = jnp.where(r == c, jnp.broadcast_to(wrow[None, :], (_CAP, _CAP)), 0.0)
    scaled = jnp.dot(diag, delta, preferred_element_type=jnp.float32)
    out_ref[...] = jnp.where(i == _NUM_EXPERT, 0.0, scaled)


def _expert(disp, expert_w, wgt_tbl, expert_b):
    wgt3 = wgt_tbl.reshape(_NUM_EXPERT, 1, _CAP)
    b3 = expert_b.reshape(_NUM_EXPERT, 1, _D)
    last = _NUM_EXPERT - 1
    return pl.pallas_call(
        _expert_body,
        grid=(_NUM_EXPERT + 1,),
        in_specs=[
            pl.BlockSpec((_CAP, _D), lambda i: (jnp.minimum(i, last), 0)),
            pl.BlockSpec((1, _D, _D), lambda i: (jnp.minimum(i, last), 0, 0)),
            pl.BlockSpec((1, 1, _CAP), lambda i: (jnp.minimum(i, last), 0, 0)),
            pl.BlockSpec((1, 1, _D), lambda i: (jnp.minimum(i, last), 0, 0)),
        ],
        out_specs=pl.BlockSpec((_CAP, _D), lambda i: (i, 0)),
        out_shape=jax.ShapeDtypeStruct(((_NUM_EXPERT + 1) * _CAP, _D),
                                       jnp.float32),
    )(disp, expert_w, wgt3, b3)


def _sc_gather(x, idx_tbl):
    N, D = x.shape
    per_w = _SLOTS // _NW
    CH = 64
    mesh = plsc.VectorSubcoreMesh(core_axis_name="c", subcore_axis_name="s")

    @functools.partial(
        pl.kernel, mesh=mesh,
        out_type=jax.ShapeDtypeStruct((_SLOTS, D), jnp.float32),
        scratch_types=[
            pltpu.VMEM((CH,), jnp.int32),
            pltpu.VMEM((CH, D), jnp.float32),
            pltpu.SemaphoreType.DMA,
        ],
    )
    def k(x_hbm, idx_hbm, out_hbm, idx_v, rows_v, sem):
        wid = lax.axis_index("s") * _NC + lax.axis_index("c")

        def body(c, carry):
            base = wid * per_w + c * CH
            pltpu.sync_copy(idx_hbm.at[pl.ds(base, CH)], idx_v)
            pltpu.async_copy(x_hbm.at[idx_v], rows_v, sem).wait()
            pltpu.sync_copy(rows_v, out_hbm.at[pl.ds(base, CH)])
            return carry

        lax.fori_loop(0, per_w // CH, body, 0)

    return k(x, idx_tbl)


def _sc_gather_pair(delta, p0, p1):
    N = p0.shape[0]
    D = delta.shape[1]
    per_w = N // _NW
    CH = 32
    NCH = per_w // CH
    mesh = plsc.VectorSubcoreMesh(core_axis_name="c", subcore_axis_name="s")

    @functools.partial(
        pl.kernel, mesh=mesh,
        out_type=[
            jax.ShapeDtypeStruct((N, D), jnp.float32),
            jax.ShapeDtypeStruct((N, D), jnp.float32),
        ],
        scratch_types=[
            pltpu.VMEM((2, CH), jnp.int32),
            pltpu.VMEM((2, CH), jnp.int32),
            pltpu.VMEM((2, CH, D), jnp.float32),
            pltpu.VMEM((2, CH, D), jnp.float32),
            pltpu.SemaphoreType.DMA,
            pltpu.SemaphoreType.DMA,
        ],
    )
    def k(delta_hbm, p0_hbm, p1_hbm, d0_hbm, d1_hbm,
          p0v, p1v, d0v, d1v, sem_a, sem_b):
        wid = lax.axis_index("s") * _NC + lax.axis_index("c")
        base = wid * per_w
        sems = (sem_a, sem_b)

        def fetch(c, b):
            tok0 = base + c * CH
            pltpu.sync_copy(p0_hbm.at[pl.ds(tok0, CH)], p0v.at[b])
            pltpu.sync_copy(p1_hbm.at[pl.ds(tok0, CH)], p1v.at[b])
            pltpu.async_copy(delta_hbm.at[p0v.at[b]], d0v.at[b], sems[b])
            pltpu.async_copy(delta_hbm.at[p1v.at[b]], d1v.at[b], sems[b])

        def drain(c, b):
            tok0 = base + c * CH
            pltpu.make_async_copy(delta_hbm.at[p0v.at[b]], d0v.at[b],
                                  sems[b]).wait()
            pltpu.make_async_copy(delta_hbm.at[p1v.at[b]], d1v.at[b],
                                  sems[b]).wait()
            pltpu.sync_copy(d0v.at[b], d0_hbm.at[pl.ds(tok0, CH)])
            pltpu.sync_copy(d1v.at[b], d1_hbm.at[pl.ds(tok0, CH)])

        fetch(0, 0)

        def pair(i, carry):
            c = i * 2

            @pl.when(c + 1 < NCH)
            def _():
                fetch(c + 1, 1)

            drain(c, 0)

            @pl.when(c + 2 < NCH)
            def _():
                fetch(c + 2, 0)

            @pl.when(c + 1 < NCH)
            def _():
                drain(c + 1, 1)

            return carry

        lax.fori_loop(0, (NCH + 1) // 2, pair, 0)

    return k(delta, p0, p1)


def _combine_body(x_ref, d0_ref, d1_ref, o_ref):
    o_ref[...] = x_ref[...] + d0_ref[...] + d1_ref[...]


def _combine(x, d0, d1):
    N, D = x.shape
    B = 1024
    spec = pl.BlockSpec((B, D), lambda i: (i, 0))
    return pl.pallas_call(
        _combine_body,
        grid=(N // B,),
        in_specs=[spec, spec, spec],
        out_specs=spec,
        out_shape=jax.ShapeDtypeStruct((N, D), jnp.float32),
    )(x, d0, d1)


def _route(g0, e0, e1):
    N = g0.shape[0]
    n_slots = N * 2
    slot_expert = jnp.stack([e0, e1], axis=1).reshape(-1)
    slot_score = jnp.stack([g0, g0], axis=1).reshape(-1)
    sort_key = slot_expert.astype(jnp.float32) * 4.0 - slot_score
    iota = jnp.arange(n_slots, dtype=jnp.int32)
    sorted_key, order = lax.sort((sort_key, iota), num_keys=1, is_stable=True)
    inv = jnp.argsort(order).astype(jnp.int32)
    sorted_expert = jnp.floor(sorted_key * 0.25).astype(jnp.int32) + 1

    e_ids = jnp.arange(_NUM_EXPERT + 1, dtype=jnp.int32)
    starts_ext = jnp.sum(
        (slot_expert[None, :] < e_ids[:, None]).astype(jnp.int32), axis=1)
    starts = starts_ext[:_NUM_EXPERT]
    counts = starts_ext[1:] - starts

    oh = (sorted_expert[:, None]
          == jnp.arange(_NUM_EXPERT, dtype=jnp.int32)[None, :]).astype(
              jnp.float32)
    starts_se = jnp.round(jnp.dot(
        oh, starts.astype(jnp.float32),
        precision=lax.Precision.HIGHEST)).astype(jnp.int32)
    rank = jnp.arange(n_slots, dtype=jnp.int32) - starts_se
    keep_sorted = rank < _CAP
    tok_sorted = (order // 2).astype(jnp.int32)
    loc_sorted = jnp.where(keep_sorted,
                           sorted_expert * _CAP + rank,
                           _SLOTS + (tok_sorted % _CAP)).astype(jnp.int32)

    pos = loc_sorted[inv]
    p0 = pos[0::2]
    p1 = pos[1::2]

    slot_g = jnp.stack([g0, 1.0 - g0], axis=1).reshape(-1)
    pr = jnp.arange(_SLOTS, dtype=jnp.int32) % _CAP
    starts_rep = jnp.broadcast_to(
        starts[:, None], (_NUM_EXPERT, _CAP)).reshape(_SLOTS)
    counts_rep = jnp.broadcast_to(
        counts[:, None], (_NUM_EXPERT, _CAP)).reshape(_SLOTS)
    j_p = lax.optimization_barrier(
        jnp.minimum(starts_rep + pr, n_slots - 1))
    valid = pr < counts_rep
    order_jp = order[j_p]
    idx_tbl = jnp.where(valid, order_jp // 2, 0)
    wgt_tbl = jnp.where(valid, slot_g[order_jp], 0.0)
    return p0, p1, idx_tbl, wgt_tbl


@jax.jit
def kernel(moe_inp, gate_w, gate_b, expert_w, expert_b):
    x = moe_inp
    g0, e0, e1 = _gate(x, gate_w, gate_b)
    p0, p1, idx_tbl, wgt_tbl = _route(g0, e0, e1)
    disp = _sc_gather(x, idx_tbl)
    delta = _expert(disp, expert_w, wgt_tbl, expert_b)
    d0, d1 = _sc_gather_pair(delta, p0, p1)
    return _combine(x, d0, d1)
